# baseline (speedup 1.0000x reference)
import functools

import jax
import jax.numpy as jnp
from jax import lax
from jax.experimental import pallas as pl
from jax.experimental.pallas import tpu as pltpu
from jax.experimental.pallas import tpu_sc as plsc

_N = 10000
_E = 320000
_D = 128
_DW = _D // 2
_L = 5

_NC = 2
_NS = 16
_NW = _NC * _NS
_EPW = _E // _NW
_CHUNK = 80
_NCHUNK = _EPW // _CHUNK
_ZROWS = 80
_ZCHUNKS = 8

_NBUF = 3
_NGRP = _NCHUNK // _NBUF
_NPEEL = _NCHUNK - _NGRP * _NBUF


def _sc_agg_body(hp_hbm, src_hbm, dst_hbm, eap_hbm, out_hbm,
                 agg_sh, idx_s, idx_d, hw, eaw,
                 sem_pre, sem_g, sem_e, sem_sc, sem_z):
    c = lax.axis_index("c")
    s = lax.axis_index("s")
    w = c * _NS + s
    base = w * _EPW

    def fire_idx(q, j):
        off = base + j * _CHUNK
        pltpu.async_copy(src_hbm.at[pl.ds(off, _CHUNK)], idx_s.at[q],
                         sem_pre.at[q])
        pltpu.async_copy(dst_hbm.at[pl.ds(off, _CHUNK)], idx_d.at[q],
                         sem_pre.at[q])

    def issue_ge(q, j):
        off = base + j * _CHUNK
        pltpu.make_async_copy(src_hbm.at[pl.ds(off, _CHUNK)], idx_s.at[q],
                              sem_pre.at[q]).wait()
        pltpu.make_async_copy(dst_hbm.at[pl.ds(off, _CHUNK)], idx_d.at[q],
                              sem_pre.at[q]).wait()
        pltpu.async_copy(eap_hbm.at[pl.ds(off, _CHUNK), :], eaw.at[q],
                         sem_e.at[q])
        pltpu.async_copy(hp_hbm.at[idx_s.at[q]], hw.at[q], sem_g.at[q])

    def wait_sc(q):
        pltpu.make_async_copy(hw.at[q], agg_sh.at[idx_d.at[q]],
                              sem_sc.at[q]).wait()

    zero16 = jnp.zeros((16,), jnp.float32)

    def zrow(r, carry):
        for j in range(_D // 16):
            hw[0, r, pl.ds(j * 16, 16)] = zero16
        return carry

    lax.fori_loop(0, _ZROWS, zrow, 0)
    for k in range(_ZCHUNKS):
        g = s * (_ZROWS * _ZCHUNKS) + k * _ZROWS

        @pl.when(g < _N)
        def _():
            pltpu.async_copy(hw.at[0], agg_sh.at[pl.ds(g, _ZROWS), :],
                             sem_z)

    for k in range(_ZCHUNKS):
        g = s * (_ZROWS * _ZCHUNKS) + k * _ZROWS

        @pl.when(g < _N)
        def _():
            pltpu.make_async_copy(hw.at[0], agg_sh.at[pl.ds(g, _ZROWS), :],
                                  sem_z).wait()

    plsc.subcore_barrier()

    def do_chunk(b, j):
        pltpu.make_async_copy(hp_hbm.at[idx_s.at[b]], hw.at[b],
                              sem_g.at[b]).wait()
        pltpu.make_async_copy(eap_hbm.at[pl.ds(0, _CHUNK), :], eaw.at[b],
                              sem_e.at[b]).wait()

        hmask = jnp.full((16,), -65536, jnp.int32)

        def rbody(r, b=b):
            for jj in range(_DW // 16):
                sl = pl.ds(jj * 16, 16)
                hwv = lax.bitcast_convert_type(hw[b, r, sl], jnp.int32)
                eav = eaw[b, r, sl]
                h_lo = lax.bitcast_convert_type(hwv << 16, jnp.float32)
                h_hi = lax.bitcast_convert_type(hwv & hmask, jnp.float32)
                e_lo = lax.bitcast_convert_type(eav << 16, jnp.float32)
                e_hi = lax.bitcast_convert_type(eav & hmask, jnp.float32)
                hw[b, r, pl.ds(jj * 16, 16)] = jnp.maximum(
                    h_lo + e_lo, 0.0)
                hw[b, r, pl.ds(_DW + jj * 16, 16)] = jnp.maximum(
                    h_hi + e_hi, 0.0)

        plsc.parallel_loop(0, _CHUNK, 1, unroll=4)(rbody)
        pltpu.async_copy(hw.at[b], agg_sh.at[idx_d.at[b]],
                         sem_sc.at[b], add=True)

        q2 = (b + 2) % _NBUF
        q1 = (b + 1) % _NBUF

        @pl.when(jnp.logical_and(j >= 1, j <= _NCHUNK - 3))
        def _():
            wait_sc(q2)

        @pl.when(j <= _NCHUNK - 3)
        def _():
            fire_idx(q2, j + 2)

        @pl.when(j <= _NCHUNK - 2)
        def _():
            issue_ge(q1, j + 1)

    fire_idx(0, 0)
    fire_idx(1, 1)
    issue_ge(0, 0)

    def group(g, carry):
        for b in range(_NBUF):
            do_chunk(b, g * _NBUF + b)
        return carry

    lax.fori_loop(0, _NGRP, group, 0)
    for t in range(_NPEEL):
        do_chunk(t, jnp.int32(_NGRP * _NBUF + t))

    for q in range(_NBUF):
        wait_sc(q)
    plsc.subcore_barrier()
    for k in range(_ZCHUNKS):
        g = s * (_ZROWS * _ZCHUNKS) + k * _ZROWS

        @pl.when(g < _N)
        def _():
            pltpu.async_copy(agg_sh.at[pl.ds(g, _ZROWS), :],
                             out_hbm.at[c, pl.ds(g, _ZROWS), :], sem_z)

    for k in range(_ZCHUNKS):
        g = s * (_ZROWS * _ZCHUNKS) + k * _ZROWS

        @pl.when(g < _N)
        def _():
            pltpu.make_async_copy(agg_sh.at[pl.ds(g, _ZROWS), :],
                                  out_hbm.at[c, pl.ds(g, _ZROWS), :],
                                  sem_z).wait()


_sc_agg = pl.kernel(
    _sc_agg_body,
    out_type=jax.ShapeDtypeStruct((_NC, _N, _D), jnp.float32),
    mesh=plsc.VectorSubcoreMesh(core_axis_name="c", subcore_axis_name="s",
                                num_cores=_NC, num_subcores=_NS),
    compiler_params=pltpu.CompilerParams(use_tc_tiling_on_sc=False),
    scratch_types=[
        pltpu.VMEM_SHARED((_N, _D), jnp.float32),
        pltpu.VMEM((_NBUF, _CHUNK), jnp.int32),
        pltpu.VMEM((_NBUF, _CHUNK), jnp.int32),
        pltpu.VMEM((_NBUF, _CHUNK, _D), jnp.float32),
        pltpu.VMEM((_NBUF, _CHUNK, _DW), jnp.int32),
        pltpu.SemaphoreType.DMA((_NBUF,)),
        pltpu.SemaphoreType.DMA((_NBUF,)),
        pltpu.SemaphoreType.DMA((_NBUF,)),
        pltpu.SemaphoreType.DMA((_NBUF,)),
        pltpu.SemaphoreType.DMA,
    ],
)


def _pack_words(x):
    lo = lax.bitcast_convert_type(
        x[:, :_DW].astype(jnp.bfloat16), jnp.uint16).astype(jnp.uint32)
    hi = lax.bitcast_convert_type(
        x[:, _DW:].astype(jnp.bfloat16), jnp.uint16).astype(jnp.uint32)
    return lax.bitcast_convert_type(lo | (hi << 16), jnp.int32)


def _pack_words_padded(x):
    w = lax.bitcast_convert_type(_pack_words(x), jnp.float32)
    return jnp.concatenate([w, jnp.zeros_like(w)], axis=1)


def _pack_body(x_ref, out_ref):
    out_ref[...] = _pack_words(x_ref[...])


def _pack_pad_body(x_ref, out_ref):
    out_ref[...] = _pack_words_padded(x_ref[...])


def _tc_pack_padded(x, bn):
    n = x.shape[0]
    return pl.pallas_call(
        _pack_pad_body,
        grid=(n // bn,),
        in_specs=[pl.BlockSpec((bn, _D), lambda i: (i, 0))],
        out_specs=pl.BlockSpec((bn, _D), lambda i: (i, 0)),
        out_shape=jax.ShapeDtypeStruct((n, _D), jnp.float32),
    )(x)


def _tc_pack(x, bn):
    n = x.shape[0]
    return pl.pallas_call(
        _pack_body,
        grid=(n // bn,),
        in_specs=[pl.BlockSpec((bn, _D), lambda i: (i, 0))],
        out_specs=pl.BlockSpec((bn, _DW), lambda i: (i, 0)),
        out_shape=jax.ShapeDtypeStruct((n, _DW), jnp.int32),
    )(x)


def _mlp_body(scale_ref, h_ref, agg_ref, w1_ref, b1_ref, w2_ref, b2_ref,
              out_ref, outp_ref):
    t = scale_ref[0] * h_ref[...] + agg_ref[0] + agg_ref[1]
    t = jnp.dot(t, w1_ref[...], preferred_element_type=jnp.float32)
    t = jnp.maximum(t + b1_ref[...], 0.0)
    t = jnp.dot(t, w2_ref[...], preferred_element_type=jnp.float32)
    h = jnp.maximum(t + b2_ref[...], 0.0)
    out_ref[...] = h
    outp_ref[...] = _pack_words_padded(h)


_BN = 1000


def _tc_mlp(h, agg, w1, b1, w2, b2, eps_l):
    scale = (1.0 + eps_l).reshape(1)
    return pl.pallas_call(
        _mlp_body,
        grid=(_N // _BN,),
        in_specs=[
            pl.BlockSpec(memory_space=pltpu.SMEM),
            pl.BlockSpec((_BN, _D), lambda i: (i, 0)),
            pl.BlockSpec((_NC, _BN, _D), lambda i: (0, i, 0)),
            pl.BlockSpec((_D, _D), lambda i: (0, 0)),
            pl.BlockSpec((1, _D), lambda i: (0, 0)),
            pl.BlockSpec((_D, _D), lambda i: (0, 0)),
            pl.BlockSpec((1, _D), lambda i: (0, 0)),
        ],
        out_specs=[
            pl.BlockSpec((_BN, _D), lambda i: (i, 0)),
            pl.BlockSpec((_BN, _D), lambda i: (i, 0)),
        ],
        out_shape=[
            jax.ShapeDtypeStruct((_N, _D), jnp.float32),
            jax.ShapeDtypeStruct((_N, _D), jnp.float32),
        ],
    )(scale, h, agg, w1, b1.reshape(1, _D), w2, b2.reshape(1, _D))


def kernel(x, edge_index, edge_attr, W1, b1, W2, b2, eps):
    src = edge_index[0]
    dst = edge_index[1]
    eap = _tc_pack(edge_attr, 3200)
    h = x
    hp = _tc_pack_padded(x, 2000)
    for l in range(_L):
        agg = _sc_agg(hp, src, dst, eap)
        h, hp = _tc_mlp(h, agg, W1[l], b1[l], W2[l], b2[l], eps[l])
    return h


# speedup vs baseline: 1.3955x; 1.3955x over previous
import functools

import jax
import jax.numpy as jnp
from jax import lax
from jax.experimental import pallas as pl
from jax.experimental.pallas import tpu as pltpu
from jax.experimental.pallas import tpu_sc as plsc

_N = 10000
_E = 320000
_D = 128
_L = 5

_NC = 2
_NS = 16
_NW = _NC * _NS
_EPW = _E // _NW
_CHUNK = 40
_NCHUNK = _EPW // _CHUNK
_ZROWS = 40
_ZCHUNKS = 16

_NBUF = 4
_HEAD = 4
_TAIL = 6
_NGRP = (_NCHUNK - _HEAD - _TAIL) // _NBUF


def _sc_agg_body(h_hbm, src_hbm, dst_hbm, ea_hbm, out_hbm,
                 agg_sh, idx_s, idx_d, rows, ea,
                 sem_pre, sem_g, sem_e, sem_sc, sem_z):
    c = lax.axis_index("c")
    s = lax.axis_index("s")
    w = c * _NS + s
    base = w * _EPW

    def fire_idx(q, j):
        off = base + j * _CHUNK
        pltpu.async_copy(src_hbm.at[pl.ds(off, _CHUNK)], idx_s.at[q],
                         sem_pre.at[q])
        pltpu.async_copy(dst_hbm.at[pl.ds(off, _CHUNK)], idx_d.at[q],
                         sem_pre.at[q])

    def issue_ge(q, j):
        off = base + j * _CHUNK
        pltpu.make_async_copy(src_hbm.at[pl.ds(off, _CHUNK)], idx_s.at[q],
                              sem_pre.at[q]).wait()
        pltpu.make_async_copy(dst_hbm.at[pl.ds(off, _CHUNK)], idx_d.at[q],
                              sem_pre.at[q]).wait()
        pltpu.async_copy(ea_hbm.at[pl.ds(off, _CHUNK), :], ea.at[q],
                         sem_e.at[q])
        pltpu.async_copy(h_hbm.at[idx_s.at[q]], rows.at[q], sem_g.at[q])

    def wait_sc(q):
        pltpu.make_async_copy(rows.at[q], agg_sh.at[idx_d.at[q]],
                              sem_sc.at[q]).wait()

    zero16 = jnp.zeros((16,), jnp.float32)

    def zrow(r, carry):
        for j in range(_D // 16):
            ea[0, r, pl.ds(j * 16, 16)] = zero16
        return carry

    lax.fori_loop(0, _ZROWS, zrow, 0)
    for k in range(_ZCHUNKS):
        g = s * (_ZROWS * _ZCHUNKS) + k * _ZROWS

        @pl.when(g < _N)
        def _():
            pltpu.async_copy(ea.at[0], agg_sh.at[pl.ds(g, _ZROWS), :],
                             sem_z)

    for k in range(_ZCHUNKS):
        g = s * (_ZROWS * _ZCHUNKS) + k * _ZROWS

        @pl.when(g < _N)
        def _():
            pltpu.make_async_copy(ea.at[0], agg_sh.at[pl.ds(g, _ZROWS), :],
                                  sem_z).wait()

    plsc.subcore_barrier()

    def do_chunk(b, j, wait3, fire, issue):
        pltpu.make_async_copy(h_hbm.at[idx_s.at[b]], rows.at[b],
                              sem_g.at[b]).wait()
        pltpu.make_async_copy(ea_hbm.at[pl.ds(0, _CHUNK), :], ea.at[b],
                              sem_e.at[b]).wait()

        def rbody(r, b=b):
            for jj in range(_D // 16):
                sl = pl.ds(jj * 16, 16)
                rows[b, r, sl] = jnp.maximum(rows[b, r, sl] + ea[b, r, sl],
                                             0.0)

        plsc.parallel_loop(0, _CHUNK, 1, unroll=4)(rbody)
        pltpu.async_copy(rows.at[b], agg_sh.at[idx_d.at[b]],
                         sem_sc.at[b], add=True)

        q3 = (b + 3) % _NBUF
        q2 = (b + 2) % _NBUF
        if wait3:
            wait_sc(q3)
        if fire:
            fire_idx(q3, j + 3)
        if issue:
            issue_ge(q2, j + 2)

    fire_idx(0, 0)
    fire_idx(1, 1)
    fire_idx(2, 2)
    issue_ge(0, 0)
    issue_ge(1, 1)

    for j in range(_HEAD):
        do_chunk(j % _NBUF, jnp.int32(j), wait3=(j >= 1), fire=True,
                 issue=True)

    def group(g, carry):
        for b in range(_NBUF):
            do_chunk(b, _HEAD + g * _NBUF + b, wait3=True, fire=True,
                     issue=True)
        return carry

    lax.fori_loop(0, _NGRP, group, 0)

    for j in range(_NCHUNK - _TAIL, _NCHUNK):
        do_chunk(j % _NBUF, jnp.int32(j),
                 wait3=(1 <= j <= _NCHUNK - 4),
                 fire=(j <= _NCHUNK - 4),
                 issue=(j <= _NCHUNK - 3))

    for q in range(_NBUF):
        wait_sc(q)
    plsc.subcore_barrier()
    for k in range(_ZCHUNKS):
        g = s * (_ZROWS * _ZCHUNKS) + k * _ZROWS

        @pl.when(g < _N)
        def _():
            pltpu.async_copy(agg_sh.at[pl.ds(g, _ZROWS), :],
                             out_hbm.at[c, pl.ds(g, _ZROWS), :], sem_z)

    for k in range(_ZCHUNKS):
        g = s * (_ZROWS * _ZCHUNKS) + k * _ZROWS

        @pl.when(g < _N)
        def _():
            pltpu.make_async_copy(agg_sh.at[pl.ds(g, _ZROWS), :],
                                  out_hbm.at[c, pl.ds(g, _ZROWS), :],
                                  sem_z).wait()


_sc_agg = pl.kernel(
    _sc_agg_body,
    out_type=jax.ShapeDtypeStruct((_NC, _N, _D), jnp.float32),
    mesh=plsc.VectorSubcoreMesh(core_axis_name="c", subcore_axis_name="s",
                                num_cores=_NC, num_subcores=_NS),
    scratch_types=[
        pltpu.VMEM_SHARED((_N, _D), jnp.float32),
        pltpu.VMEM((_NBUF, _CHUNK), jnp.int32),
        pltpu.VMEM((_NBUF, _CHUNK), jnp.int32),
        pltpu.VMEM((_NBUF, _CHUNK, _D), jnp.float32),
        pltpu.VMEM((_NBUF, _CHUNK, _D), jnp.float32),
        pltpu.SemaphoreType.DMA((_NBUF,)),
        pltpu.SemaphoreType.DMA((_NBUF,)),
        pltpu.SemaphoreType.DMA((_NBUF,)),
        pltpu.SemaphoreType.DMA((_NBUF,)),
        pltpu.SemaphoreType.DMA,
    ],
)


def _mlp_body(scale_ref, h_ref, agg_ref, w1_ref, b1_ref, w2_ref, b2_ref,
              out_ref):
    t = scale_ref[0] * h_ref[...] + agg_ref[0] + agg_ref[1]
    t = jnp.dot(t, w1_ref[...], preferred_element_type=jnp.float32)
    t = jnp.maximum(t + b1_ref[...], 0.0)
    t = jnp.dot(t, w2_ref[...], preferred_element_type=jnp.float32)
    out_ref[...] = jnp.maximum(t + b2_ref[...], 0.0)


_BN = 1000


def _tc_mlp(h, agg, w1, b1, w2, b2, eps_l):
    scale = (1.0 + eps_l).reshape(1)
    return pl.pallas_call(
        _mlp_body,
        grid=(_N // _BN,),
        in_specs=[
            pl.BlockSpec(memory_space=pltpu.SMEM),
            pl.BlockSpec((_BN, _D), lambda i: (i, 0)),
            pl.BlockSpec((_NC, _BN, _D), lambda i: (0, i, 0)),
            pl.BlockSpec((_D, _D), lambda i: (0, 0)),
            pl.BlockSpec((1, _D), lambda i: (0, 0)),
            pl.BlockSpec((_D, _D), lambda i: (0, 0)),
            pl.BlockSpec((1, _D), lambda i: (0, 0)),
        ],
        out_specs=pl.BlockSpec((_BN, _D), lambda i: (i, 0)),
        out_shape=jax.ShapeDtypeStruct((_N, _D), jnp.float32),
    )(scale, h, agg, w1, b1.reshape(1, _D), w2, b2.reshape(1, _D))


def kernel(x, edge_index, edge_attr, W1, b1, W2, b2, eps):
    src = edge_index[0]
    dst = edge_index[1]
    h = x
    for l in range(_L):
        agg = _sc_agg(h, src, dst, edge_attr)
        h = _tc_mlp(h, agg, W1[l], b1[l], W2[l], b2[l], eps[l])
    return h


# speedup vs baseline: 1.4113x; 1.0114x over previous
import functools

import jax
import jax.numpy as jnp
from jax import lax
from jax.experimental import pallas as pl
from jax.experimental.pallas import tpu as pltpu
from jax.experimental.pallas import tpu_sc as plsc

_N = 10000
_E = 320000
_D = 128
_L = 5

_NC = 2
_NS = 16
_NW = _NC * _NS
_EPW = _E // _NW
_CHUNK = 40
_NCHUNK = _EPW // _CHUNK
_ZROWS = 40
_ZCHUNKS = 16

_NBUF = 4
_HEAD = 4
_TAIL = 6
_NGRP = (_NCHUNK - _HEAD - _TAIL) // _NBUF


def _sc_agg_body(h_hbm, ei_hbm, ea_hbm, out_hbm,
                 agg_sh, idx, rows, ea,
                 sem_pre, sem_g, sem_e, sem_sc, sem_z):
    c = lax.axis_index("c")
    s = lax.axis_index("s")
    w = c * _NS + s
    base = w * _EPW

    def fire_idx(q, j):
        off = base + j * _CHUNK
        pltpu.async_copy(ei_hbm.at[:, pl.ds(off, _CHUNK)], idx.at[q],
                         sem_pre.at[q])

    def issue_ge(q, j):
        off = base + j * _CHUNK
        pltpu.make_async_copy(ei_hbm.at[:, pl.ds(off, _CHUNK)], idx.at[q],
                              sem_pre.at[q]).wait()
        pltpu.async_copy(ea_hbm.at[pl.ds(off, _CHUNK), :], ea.at[q],
                         sem_e.at[q])
        pltpu.async_copy(h_hbm.at[idx.at[q, 0]], rows.at[q], sem_g.at[q])

    def wait_sc(q):
        pltpu.make_async_copy(rows.at[q], agg_sh.at[idx.at[q, 1]],
                              sem_sc.at[q]).wait()

    zero16 = jnp.zeros((16,), jnp.float32)

    def zrow(r, carry):
        for j in range(_D // 16):
            ea[0, r, pl.ds(j * 16, 16)] = zero16
        return carry

    lax.fori_loop(0, _ZROWS, zrow, 0)
    for k in range(_ZCHUNKS):
        g = s * (_ZROWS * _ZCHUNKS) + k * _ZROWS

        @pl.when(g < _N)
        def _():
            pltpu.async_copy(ea.at[0], agg_sh.at[pl.ds(g, _ZROWS), :],
                             sem_z)

    for k in range(_ZCHUNKS):
        g = s * (_ZROWS * _ZCHUNKS) + k * _ZROWS

        @pl.when(g < _N)
        def _():
            pltpu.make_async_copy(ea.at[0], agg_sh.at[pl.ds(g, _ZROWS), :],
                                  sem_z).wait()

    plsc.subcore_barrier()

    def do_chunk(b, j, wait3, fire, issue):
        pltpu.make_async_copy(h_hbm.at[idx.at[b, 0]], rows.at[b],
                              sem_g.at[b]).wait()
        pltpu.make_async_copy(ea_hbm.at[pl.ds(0, _CHUNK), :], ea.at[b],
                              sem_e.at[b]).wait()

        def rbody(r, b=b):
            for jj in range(_D // 16):
                sl = pl.ds(jj * 16, 16)
                rows[b, r, sl] = jnp.maximum(rows[b, r, sl] + ea[b, r, sl],
                                             0.0)

        plsc.parallel_loop(0, _CHUNK, 1, unroll=4)(rbody)
        pltpu.async_copy(rows.at[b], agg_sh.at[idx.at[b, 1]],
                         sem_sc.at[b], add=True)

        q3 = (b + 3) % _NBUF
        q2 = (b + 2) % _NBUF
        if wait3:
            wait_sc(q3)
        if fire:
            fire_idx(q3, j + 3)
        if issue:
            issue_ge(q2, j + 2)

    fire_idx(0, 0)
    fire_idx(1, 1)
    fire_idx(2, 2)
    issue_ge(0, 0)
    issue_ge(1, 1)

    for j in range(_HEAD):
        do_chunk(j % _NBUF, jnp.int32(j), wait3=(j >= 1), fire=True,
                 issue=True)

    def group(g, carry):
        for b in range(_NBUF):
            do_chunk(b, _HEAD + g * _NBUF + b, wait3=True, fire=True,
                     issue=True)
        return carry

    lax.fori_loop(0, _NGRP, group, 0)

    for j in range(_NCHUNK - _TAIL, _NCHUNK):
        do_chunk(j % _NBUF, jnp.int32(j),
                 wait3=(1 <= j <= _NCHUNK - 4),
                 fire=(j <= _NCHUNK - 4),
                 issue=(j <= _NCHUNK - 3))

    for q in range(_NBUF):
        wait_sc(q)
    plsc.subcore_barrier()
    for k in range(_ZCHUNKS):
        g = s * (_ZROWS * _ZCHUNKS) + k * _ZROWS

        @pl.when(g < _N)
        def _():
            pltpu.async_copy(agg_sh.at[pl.ds(g, _ZROWS), :],
                             out_hbm.at[c, pl.ds(g, _ZROWS), :], sem_z)

    for k in range(_ZCHUNKS):
        g = s * (_ZROWS * _ZCHUNKS) + k * _ZROWS

        @pl.when(g < _N)
        def _():
            pltpu.make_async_copy(agg_sh.at[pl.ds(g, _ZROWS), :],
                                  out_hbm.at[c, pl.ds(g, _ZROWS), :],
                                  sem_z).wait()


_sc_agg = pl.kernel(
    _sc_agg_body,
    out_type=jax.ShapeDtypeStruct((_NC, _N, _D), jnp.float32),
    mesh=plsc.VectorSubcoreMesh(core_axis_name="c", subcore_axis_name="s",
                                num_cores=_NC, num_subcores=_NS),
    compiler_params=pltpu.CompilerParams(use_tc_tiling_on_sc=False),
    scratch_types=[
        pltpu.VMEM_SHARED((_N, _D), jnp.float32),
        pltpu.VMEM((_NBUF, 2, _CHUNK), jnp.int32),
        pltpu.VMEM((_NBUF, _CHUNK, _D), jnp.float32),
        pltpu.VMEM((_NBUF, _CHUNK, _D), jnp.float32),
        pltpu.SemaphoreType.DMA((_NBUF,)),
        pltpu.SemaphoreType.DMA((_NBUF,)),
        pltpu.SemaphoreType.DMA((_NBUF,)),
        pltpu.SemaphoreType.DMA((_NBUF,)),
        pltpu.SemaphoreType.DMA,
    ],
)


def _mlp_body(scale_ref, h_ref, agg_ref, w1_ref, b1_ref, w2_ref, b2_ref,
              out_ref):
    t = scale_ref[0] * h_ref[...] + agg_ref[0] + agg_ref[1]
    t = jnp.dot(t, w1_ref[...], preferred_element_type=jnp.float32)
    t = jnp.maximum(t + b1_ref[...], 0.0)
    t = jnp.dot(t, w2_ref[...], preferred_element_type=jnp.float32)
    out_ref[...] = jnp.maximum(t + b2_ref[...], 0.0)


_BN = 1000


def _tc_mlp(h, agg, w1, b1, w2, b2, eps_l):
    scale = (1.0 + eps_l).reshape(1)
    return pl.pallas_call(
        _mlp_body,
        grid=(_N // _BN,),
        in_specs=[
            pl.BlockSpec(memory_space=pltpu.SMEM),
            pl.BlockSpec((_BN, _D), lambda i: (i, 0)),
            pl.BlockSpec((_NC, _BN, _D), lambda i: (0, i, 0)),
            pl.BlockSpec((_D, _D), lambda i: (0, 0)),
            pl.BlockSpec((1, _D), lambda i: (0, 0)),
            pl.BlockSpec((_D, _D), lambda i: (0, 0)),
            pl.BlockSpec((1, _D), lambda i: (0, 0)),
        ],
        out_specs=pl.BlockSpec((_BN, _D), lambda i: (i, 0)),
        out_shape=jax.ShapeDtypeStruct((_N, _D), jnp.float32),
    )(scale, h, agg, w1, b1.reshape(1, _D), w2, b2.reshape(1, _D))


def kernel(x, edge_index, edge_attr, W1, b1, W2, b2, eps):
    h = x
    for l in range(_L):
        agg = _sc_agg(h, edge_index, edge_attr)
        h = _tc_mlp(h, agg, W1[l], b1[l], W2[l], b2[l], eps[l])
    return h
